# Initial kernel scaffold; baseline (speedup 1.0000x reference)
import jax
import jax.numpy as jnp
from jax.experimental import pallas as pl


def kernel(x_3d, x_mod_feat, fm_idx, atomic_seg, view_seg, W_conv, b_conv):
    raise NotImplementedError("write your pallas kernel here")


# speedup vs baseline: 2.2736x; 2.2736x over previous
import functools

import jax
import jax.numpy as jnp
from jax import lax
from jax.experimental import pallas as pl
from jax.experimental.pallas import tpu as pltpu
from jax.experimental.pallas import tpu_sc as plsc

_N_POINTS = 10000
_N_PIXELS = 50000
_N_MAP = 320000
_N_VIEWS = 40000
_D = 128
_L = 16
_DB = _D // _L

_NC, _NS = 2, 16
_NW = _NC * _NS

_T1 = 256
_NV_PAD = 40960
_NT1 = _NV_PAD // _T1
_TPW1 = _NT1 // _NW
_K1 = 128

_K2 = 256
_PPW = 320
_NPTS_PAD = _PPW * _NW

_NEG_INF = float("-inf")


def _lane0():
    return jnp.where(lax.iota(jnp.int32, _L) == 0,
                     jnp.float32(1.0), jnp.float32(0.0))


def _sload(ref, i):
    return ref[pl.ds(i, _L)][0]


def _conv_body(x_ref, w_ref, b_ref, o_ref):
    o_ref[...] = (
        jnp.dot(x_ref[...], w_ref[...], preferred_element_type=jnp.float32)
        + b_ref[...]
    )


def _conv(x, w, b):
    m = x.shape[0]
    bm = 2000
    return pl.pallas_call(
        _conv_body,
        grid=(m // bm,),
        in_specs=[
            pl.BlockSpec((bm, _D), lambda i: (i, 0)),
            pl.BlockSpec((_D, _D), lambda i: (0, 0)),
            pl.BlockSpec((1, _D), lambda i: (0, 0)),
        ],
        out_specs=pl.BlockSpec((bm, _D), lambda i: (i, 0)),
        out_shape=jax.ShapeDtypeStruct((m, _D), jnp.float32),
    )(x, w, b.reshape(1, _D))


def _sc_mesh():
    return plsc.VectorSubcoreMesh(core_axis_name="c", subcore_axis_name="s")


def _stage1(x_mod, fm_idx_pad, seg_pad, rs_pad):
    @functools.partial(
        pl.kernel,
        out_type=jax.ShapeDtypeStruct((_NV_PAD + _K2, _D), jnp.float32),
        mesh=_sc_mesh(),
        scratch_types=[
            pltpu.VMEM((rs_pad.shape[0] + _L,), jnp.int32),
            pltpu.VMEM((_K1,), jnp.int32),
            pltpu.VMEM((_K1 + _L,), jnp.int32),
            pltpu.VMEM((_K1, _D), jnp.float32),
            pltpu.VMEM((_T1, _D), jnp.float32),
            pltpu.SemaphoreType.DMA,
        ],
    )
    def k(xmod_hbm, fmidx_hbm, seg_hbm, rs_hbm, out_hbm,
          rs_v, idx_v, seg_v, rows_v, acc_v, sem):
        wid = lax.axis_index("s") * _NC + lax.axis_index("c")
        pltpu.sync_copy(rs_hbm, rs_v.at[pl.ds(0, rs_hbm.shape[0])])

        for t in range(_TPW1):
            tile = wid * _TPW1 + t
            vbase = tile * _T1
            r0 = _sload(rs_v, tile)
            r1 = _sload(rs_v, tile + 1)
            a0 = (r0 // 8) * 8

            @pl.loop(0, _T1)
            def _init(v):
                for j in range(_DB):
                    acc_v[v, pl.ds(j * _L, _L)] = jnp.full(
                        (_L,), _NEG_INF, jnp.float32)

            @pl.loop(a0, r1, step=_K1)
            def _chunk(pos):
                pltpu.sync_copy(fmidx_hbm.at[pl.ds(pos, _K1)], idx_v)
                pltpu.sync_copy(
                    seg_hbm.at[pl.ds(pos, _K1)], seg_v.at[pl.ds(0, _K1)])
                pltpu.async_copy(xmod_hbm.at[idx_v], rows_v, sem).wait()
                lo = jnp.maximum(r0 - pos, 0)
                hi = jnp.minimum(r1 - pos, _K1)

                @pl.loop(lo, hi)
                def _row(i):
                    rel = _sload(seg_v, i) - vbase
                    for j in range(_DB):
                        sl = pl.ds(j * _L, _L)
                        acc_v[rel, sl] = jnp.maximum(
                            acc_v[rel, sl], rows_v[i, sl])

            @pl.loop(0, _T1)
            def _fin(v):
                for j in range(_DB):
                    sl = pl.ds(j * _L, _L)
                    x = acc_v[v, sl]
                    acc_v[v, sl] = jnp.where(
                        x == _NEG_INF, jnp.float32(0.0), x)

            pltpu.sync_copy(acc_v, out_hbm.at[pl.ds(vbase, _T1)])

    return k(x_mod, fm_idx_pad, seg_pad, rs_pad)


def _stage2(x_atomic, vseg_pad, rs_pad):
    @functools.partial(
        pl.kernel,
        out_type=(
            jax.ShapeDtypeStruct((_NPTS_PAD, _D), jnp.float32),
            jax.ShapeDtypeStruct((_NPTS_PAD,), jnp.float32),
        ),
        mesh=_sc_mesh(),
        scratch_types=[
            pltpu.VMEM((rs_pad.shape[0] + _L,), jnp.int32),
            pltpu.VMEM((_K2 + _L,), jnp.int32),
            pltpu.VMEM((_K2, _D), jnp.float32),
            pltpu.VMEM((_PPW, _D), jnp.float32),
            pltpu.VMEM((_PPW + _L,), jnp.float32),
            pltpu.SemaphoreType.DMA,
        ],
    )
    def k(xa_hbm, vseg_hbm, rs_hbm, sums_hbm, cnts_hbm,
          rs_v, seg_v, rows_v, sum_v, cnt_v, sem):
        wid = lax.axis_index("s") * _NC + lax.axis_index("c")
        pltpu.sync_copy(rs_hbm, rs_v.at[pl.ds(0, rs_hbm.shape[0])])
        p0 = wid * _PPW
        r0 = _sload(rs_v, wid)
        r1 = _sload(rs_v, wid + 1)
        a0 = (r0 // 8) * 8

        @pl.loop(0, _PPW + _L, step=_L)
        def _initc(v):
            cnt_v[pl.ds(v, _L)] = jnp.zeros((_L,), jnp.float32)

        @pl.loop(0, _PPW)
        def _init(v):
            for j in range(_DB):
                sum_v[v, pl.ds(j * _L, _L)] = jnp.zeros((_L,), jnp.float32)

        @pl.loop(a0, r1, step=_K2)
        def _chunk(pos):
            pltpu.sync_copy(
                vseg_hbm.at[pl.ds(pos, _K2)], seg_v.at[pl.ds(0, _K2)])
            pltpu.async_copy(xa_hbm.at[pl.ds(pos, _K2)], rows_v, sem).wait()
            lo = jnp.maximum(r0 - pos, 0)
            hi = jnp.minimum(r1 - pos, _K2)

            @pl.loop(lo, hi)
            def _row(i):
                rel = _sload(seg_v, i) - p0
                cwin = cnt_v[pl.ds(rel, _L)]
                cnt_v[pl.ds(rel, _L)] = cwin + _lane0()
                for j in range(_DB):
                    sl = pl.ds(j * _L, _L)
                    sum_v[rel, sl] = sum_v[rel, sl] + rows_v[i, sl]

        pltpu.sync_copy(sum_v, sums_hbm.at[pl.ds(p0, _PPW)])
        pltpu.sync_copy(cnt_v.at[pl.ds(0, _PPW)], cnts_hbm.at[pl.ds(p0, _PPW)])

    return k(x_atomic, vseg_pad, rs_pad)


def _finish_body(x3_ref, s_ref, c_ref, o_ref):
    inv = jnp.float32(1.0) / jnp.maximum(c_ref[...], jnp.float32(1.0))
    o_ref[...] = x3_ref[...] + s_ref[...] * inv


def _finish(x3_pad, sums, cnts):
    bm = 1280
    return pl.pallas_call(
        _finish_body,
        grid=(_NPTS_PAD // bm,),
        in_specs=[
            pl.BlockSpec((bm, _D), lambda i: (i, 0)),
            pl.BlockSpec((bm, _D), lambda i: (i, 0)),
            pl.BlockSpec((bm, 1), lambda i: (i, 0)),
        ],
        out_specs=pl.BlockSpec((bm, _D), lambda i: (i, 0)),
        out_shape=jax.ShapeDtypeStruct((_NPTS_PAD, _D), jnp.float32),
    )(x3_pad, sums, cnts.reshape(_NPTS_PAD, 1))


def kernel(x_3d, x_mod_feat, fm_idx, atomic_seg, view_seg, W_conv, b_conv):
    fm = fm_idx.astype(jnp.int32)
    aseg = atomic_seg.astype(jnp.int32)
    vseg = view_seg.astype(jnp.int32)

    x_mod = _conv(x_mod_feat, W_conv, b_conv)

    vb1 = jnp.arange(0, _NV_PAD + 1, _T1, dtype=jnp.int32)
    rs1 = jnp.searchsorted(aseg, vb1).astype(jnp.int32)
    rs1 = jnp.concatenate([rs1, jnp.full((7,), _N_MAP, jnp.int32)])
    fm_pad = jnp.concatenate([fm, jnp.zeros((_K1,), jnp.int32)])
    aseg_pad = jnp.concatenate([aseg, jnp.full((_K1,), _N_VIEWS, jnp.int32)])

    x_atomic = _stage1(x_mod, fm_pad, aseg_pad, rs1)

    pb = jnp.arange(0, _NPTS_PAD + 1, _PPW, dtype=jnp.int32)
    rs2 = jnp.searchsorted(vseg, pb).astype(jnp.int32)
    rs2 = jnp.concatenate([rs2, jnp.full((7,), _N_VIEWS, jnp.int32)])
    vseg_pad = jnp.concatenate([vseg, jnp.full((_K2,), _NPTS_PAD, jnp.int32)])
    x3_pad = jnp.concatenate(
        [x_3d, jnp.zeros((_NPTS_PAD - _N_POINTS, _D), jnp.float32)])

    sums, cnts = _stage2(x_atomic, vseg_pad, rs2)
    out = _finish(x3_pad, sums, cnts)
    return out[:_N_POINTS]


# speedup vs baseline: 2.7553x; 1.2119x over previous
import functools

import jax
import jax.numpy as jnp
from jax import lax
from jax.experimental import pallas as pl
from jax.experimental.pallas import tpu as pltpu
from jax.experimental.pallas import tpu_sc as plsc

_N_POINTS = 10000
_N_PIXELS = 50000
_N_MAP = 320000
_N_VIEWS = 40000
_D = 128
_L = 16
_DB = _D // _L

_NC, _NS = 2, 16
_NW = _NC * _NS

_T1 = 256
_NV_PAD = 40960
_NT1 = _NV_PAD // _T1
_TPW1 = _NT1 // _NW
_K1 = 128

_K2 = 256
_PPW = 320
_NPTS_PAD = _PPW * _NW

_NEG_INF = float("-inf")


def _lane0():
    return jnp.where(lax.iota(jnp.int32, _L) == 0,
                     jnp.float32(1.0), jnp.float32(0.0))


def _sload(ref, i):
    return ref[pl.ds(i, _L)][0]


def _conv_body(x_ref, w_ref, b_ref, o_ref):
    o_ref[...] = (
        jnp.dot(x_ref[...], w_ref[...], preferred_element_type=jnp.float32)
        + b_ref[...]
    )


def _conv(x, w, b):
    m = x.shape[0]
    bm = 2000
    return pl.pallas_call(
        _conv_body,
        grid=(m // bm,),
        in_specs=[
            pl.BlockSpec((bm, _D), lambda i: (i, 0)),
            pl.BlockSpec((_D, _D), lambda i: (0, 0)),
            pl.BlockSpec((1, _D), lambda i: (0, 0)),
        ],
        out_specs=pl.BlockSpec((bm, _D), lambda i: (i, 0)),
        out_shape=jax.ShapeDtypeStruct((m, _D), jnp.float32),
    )(x, w, b.reshape(1, _D))


def _sc_mesh():
    return plsc.VectorSubcoreMesh(core_axis_name="c", subcore_axis_name="s")


def _stage1(x_mod, fm_idx_pad, seg_pad, rs_pad):
    @functools.partial(
        pl.kernel,
        out_type=jax.ShapeDtypeStruct((_NV_PAD + _K2, _D), jnp.float32),
        mesh=_sc_mesh(),
        scratch_types=[
            pltpu.VMEM((rs_pad.shape[0] + _L,), jnp.int32),
            [pltpu.VMEM((_K1,), jnp.int32)] * 4,
            [pltpu.VMEM((_K1 + _L,), jnp.int32)] * 4,
            [pltpu.VMEM((_K1, _D), jnp.float32)] * 2,
            pltpu.VMEM((_T1, _D), jnp.float32),
            [pltpu.SemaphoreType.DMA] * 4,
            [pltpu.SemaphoreType.DMA] * 2,
        ],
    )
    def k(xmod_hbm, fmidx_hbm, seg_hbm, rs_hbm, out_hbm,
          rs_v, idx_v, seg_v, rows_v, acc_v, si, sg):
        wid = lax.axis_index("s") * _NC + lax.axis_index("c")
        pltpu.sync_copy(rs_hbm, rs_v.at[pl.ds(0, rs_hbm.shape[0])])

        for t in range(_TPW1):
            tile = wid * _TPW1 + t
            vbase = tile * _T1
            r0 = _sload(rs_v, tile)
            r1 = _sload(rs_v, tile + 1)
            a0 = (r0 // 8) * 8
            n = (r1 - a0 + _K1 - 1) // _K1

            def start_idx(c, s):
                pos = a0 + c * _K1
                pltpu.async_copy(
                    fmidx_hbm.at[pl.ds(pos, _K1)], idx_v[s], si[s])
                pltpu.async_copy(
                    seg_hbm.at[pl.ds(pos, _K1)],
                    seg_v[s].at[pl.ds(0, _K1)], si[s])

            def wait_idx(c, s):
                pos = a0 + c * _K1
                pltpu.make_async_copy(
                    fmidx_hbm.at[pl.ds(pos, _K1)], idx_v[s], si[s]).wait()
                pltpu.make_async_copy(
                    seg_hbm.at[pl.ds(pos, _K1)],
                    seg_v[s].at[pl.ds(0, _K1)], si[s]).wait()

            def start_gather(s, b):
                pltpu.async_copy(xmod_hbm.at[idx_v[s]], rows_v[b], sg[b])

            def wait_gather(s, b):
                pltpu.make_async_copy(
                    xmod_hbm.at[idx_v[s]], rows_v[b], sg[b]).wait()

            def process(c, b, s):
                pos = a0 + c * _K1
                lo = jnp.maximum(r0 - pos, 0)
                hi = jnp.minimum(r1 - pos, _K1)

                @pl.loop(lo, hi)
                def _row(i):
                    rel = _sload(seg_v[s], i) - vbase
                    for j in range(_DB):
                        sl = pl.ds(j * _L, _L)
                        acc_v[rel, sl] = jnp.maximum(
                            acc_v[rel, sl], rows_v[b][i, sl])

            @pl.loop(0, _T1)
            def _init(v):
                for j in range(_DB):
                    acc_v[v, pl.ds(j * _L, _L)] = jnp.full(
                        (_L,), _NEG_INF, jnp.float32)

            @pl.when(n > 0)
            def _p0():
                start_idx(0, 0)

            @pl.when(n > 1)
            def _p1():
                start_idx(1, 1)

            @pl.when(n > 0)
            def _p2():
                wait_idx(0, 0)
                start_gather(0, 0)

            @pl.loop(0, n, step=4)
            def _quad(c):
                for q in range(4):
                    cq = c + q
                    s, b = q % 4, q % 2
                    s1, b1 = (q + 1) % 4, (q + 1) % 2
                    s2 = (q + 2) % 4

                    @pl.when(cq < n)
                    def _(cq=cq, s=s, b=b, s1=s1, b1=b1, s2=s2):
                        wait_gather(s, b)

                        @pl.when(cq + 1 < n)
                        def _():
                            wait_idx(cq + 1, s1)
                            start_gather(s1, b1)

                        @pl.when(cq + 2 < n)
                        def _():
                            start_idx(cq + 2, s2)

                        process(cq, b, s)

            @pl.loop(0, _T1)
            def _fin(v):
                for j in range(_DB):
                    sl = pl.ds(j * _L, _L)
                    x = acc_v[v, sl]
                    acc_v[v, sl] = jnp.where(
                        x == _NEG_INF, jnp.float32(0.0), x)

            pltpu.sync_copy(acc_v, out_hbm.at[pl.ds(vbase, _T1)])

    return k(x_mod, fm_idx_pad, seg_pad, rs_pad)


def _stage2(x_atomic, vseg_pad, rs_pad):
    @functools.partial(
        pl.kernel,
        out_type=(
            jax.ShapeDtypeStruct((_NPTS_PAD, _D), jnp.float32),
            jax.ShapeDtypeStruct((_NPTS_PAD,), jnp.float32),
        ),
        mesh=_sc_mesh(),
        scratch_types=[
            pltpu.VMEM((rs_pad.shape[0] + _L,), jnp.int32),
            pltpu.VMEM((_K2 + _L,), jnp.int32),
            pltpu.VMEM((_K2, _D), jnp.float32),
            pltpu.VMEM((_PPW, _D), jnp.float32),
            pltpu.VMEM((_PPW + _L,), jnp.float32),
            pltpu.SemaphoreType.DMA,
        ],
    )
    def k(xa_hbm, vseg_hbm, rs_hbm, sums_hbm, cnts_hbm,
          rs_v, seg_v, rows_v, sum_v, cnt_v, sem):
        wid = lax.axis_index("s") * _NC + lax.axis_index("c")
        pltpu.sync_copy(rs_hbm, rs_v.at[pl.ds(0, rs_hbm.shape[0])])
        p0 = wid * _PPW
        r0 = _sload(rs_v, wid)
        r1 = _sload(rs_v, wid + 1)
        a0 = (r0 // 8) * 8

        @pl.loop(0, _PPW + _L, step=_L)
        def _initc(v):
            cnt_v[pl.ds(v, _L)] = jnp.zeros((_L,), jnp.float32)

        @pl.loop(0, _PPW)
        def _init(v):
            for j in range(_DB):
                sum_v[v, pl.ds(j * _L, _L)] = jnp.zeros((_L,), jnp.float32)

        @pl.loop(a0, r1, step=_K2)
        def _chunk(pos):
            pltpu.sync_copy(
                vseg_hbm.at[pl.ds(pos, _K2)], seg_v.at[pl.ds(0, _K2)])
            pltpu.async_copy(xa_hbm.at[pl.ds(pos, _K2)], rows_v, sem).wait()
            lo = jnp.maximum(r0 - pos, 0)
            hi = jnp.minimum(r1 - pos, _K2)

            @pl.loop(lo, hi)
            def _row(i):
                rel = _sload(seg_v, i) - p0
                cwin = cnt_v[pl.ds(rel, _L)]
                cnt_v[pl.ds(rel, _L)] = cwin + _lane0()
                for j in range(_DB):
                    sl = pl.ds(j * _L, _L)
                    sum_v[rel, sl] = sum_v[rel, sl] + rows_v[i, sl]

        pltpu.sync_copy(sum_v, sums_hbm.at[pl.ds(p0, _PPW)])
        pltpu.sync_copy(cnt_v.at[pl.ds(0, _PPW)], cnts_hbm.at[pl.ds(p0, _PPW)])

    return k(x_atomic, vseg_pad, rs_pad)


def _finish_body(x3_ref, s_ref, c_ref, o_ref):
    inv = jnp.float32(1.0) / jnp.maximum(c_ref[...], jnp.float32(1.0))
    o_ref[...] = x3_ref[...] + s_ref[...] * inv


def _finish(x3_pad, sums, cnts):
    bm = 1280
    return pl.pallas_call(
        _finish_body,
        grid=(_NPTS_PAD // bm,),
        in_specs=[
            pl.BlockSpec((bm, _D), lambda i: (i, 0)),
            pl.BlockSpec((bm, _D), lambda i: (i, 0)),
            pl.BlockSpec((bm, 1), lambda i: (i, 0)),
        ],
        out_specs=pl.BlockSpec((bm, _D), lambda i: (i, 0)),
        out_shape=jax.ShapeDtypeStruct((_NPTS_PAD, _D), jnp.float32),
    )(x3_pad, sums, cnts.reshape(_NPTS_PAD, 1))


def kernel(x_3d, x_mod_feat, fm_idx, atomic_seg, view_seg, W_conv, b_conv):
    fm = fm_idx.astype(jnp.int32)
    aseg = atomic_seg.astype(jnp.int32)
    vseg = view_seg.astype(jnp.int32)

    x_mod = _conv(x_mod_feat, W_conv, b_conv)

    vb1 = jnp.arange(0, _NV_PAD + 1, _T1, dtype=jnp.int32)
    rs1 = jnp.searchsorted(aseg, vb1).astype(jnp.int32)
    rs1 = jnp.concatenate([rs1, jnp.full((7,), _N_MAP, jnp.int32)])
    fm_pad = jnp.concatenate([fm, jnp.zeros((_K1,), jnp.int32)])
    aseg_pad = jnp.concatenate([aseg, jnp.full((_K1,), _N_VIEWS, jnp.int32)])

    x_atomic = _stage1(x_mod, fm_pad, aseg_pad, rs1)

    pb = jnp.arange(0, _NPTS_PAD + 1, _PPW, dtype=jnp.int32)
    rs2 = jnp.searchsorted(vseg, pb).astype(jnp.int32)
    rs2 = jnp.concatenate([rs2, jnp.full((7,), _N_VIEWS, jnp.int32)])
    vseg_pad = jnp.concatenate([vseg, jnp.full((_K2,), _NPTS_PAD, jnp.int32)])
    x3_pad = jnp.concatenate(
        [x_3d, jnp.zeros((_NPTS_PAD - _N_POINTS, _D), jnp.float32)])

    sums, cnts = _stage2(x_atomic, vseg_pad, rs2)
    out = _finish(x3_pad, sums, cnts)
    return out[:_N_POINTS]


# speedup vs baseline: 2.9245x; 1.0614x over previous
import functools

import jax
import jax.numpy as jnp
from jax import lax
from jax.experimental import pallas as pl
from jax.experimental.pallas import tpu as pltpu
from jax.experimental.pallas import tpu_sc as plsc

_N_POINTS = 10000
_N_PIXELS = 50000
_N_MAP = 320000
_N_VIEWS = 40000
_D = 128
_L = 16
_DB = _D // _L

_NC, _NS = 2, 16
_NW = _NC * _NS

_T1 = 640
_NV_PAD = 40960
_NT1 = _NV_PAD // _T1
_TPW1 = _NT1 // _NW
_K1 = 128

_K2 = 256
_PPW = 320
_NPTS_PAD = _PPW * _NW

_NEG_INF = float("-inf")


def _sload(ref, i):
    return ref[pl.ds(i, _L)][0]


def _conv_body(x_ref, w_ref, b_ref, o_ref):
    o_ref[...] = (
        jnp.dot(x_ref[...], w_ref[...], preferred_element_type=jnp.float32)
        + b_ref[...]
    )


def _conv(x, w, b):
    m = x.shape[0]
    bm = 2000
    return pl.pallas_call(
        _conv_body,
        grid=(m // bm,),
        in_specs=[
            pl.BlockSpec((bm, _D), lambda i: (i, 0)),
            pl.BlockSpec((_D, _D), lambda i: (0, 0)),
            pl.BlockSpec((1, _D), lambda i: (0, 0)),
        ],
        out_specs=pl.BlockSpec((bm, _D), lambda i: (i, 0)),
        out_shape=jax.ShapeDtypeStruct((m, _D), jnp.float32),
    )(x, w, b.reshape(1, _D))


def _sc_mesh():
    return plsc.VectorSubcoreMesh(core_axis_name="c", subcore_axis_name="s")


def _stage1(x_mod, fm_idx_pad, seg_pad, rs_pad):
    @functools.partial(
        pl.kernel,
        out_type=jax.ShapeDtypeStruct((_NV_PAD + _K2, _D), jnp.float32),
        mesh=_sc_mesh(),
        scratch_types=[
            pltpu.VMEM((rs_pad.shape[0] + _L,), jnp.int32),
            [pltpu.VMEM((_K1,), jnp.int32)] * 4,
            [pltpu.VMEM((_K1 + _L,), jnp.int32)] * 4,
            [pltpu.VMEM((_K1, _D), jnp.float32)] * 2,
            pltpu.VMEM((_T1, _D), jnp.float32),
            [pltpu.SemaphoreType.DMA] * 4,
            [pltpu.SemaphoreType.DMA] * 2,
        ],
    )
    def k(xmod_hbm, fmidx_hbm, seg_hbm, rs_hbm, out_hbm,
          rs_v, idx_v, seg_v, rows_v, acc_v, si, sg):
        wid = lax.axis_index("s") * _NC + lax.axis_index("c")
        pltpu.sync_copy(rs_hbm, rs_v.at[pl.ds(0, rs_hbm.shape[0])])

        @pl.loop(0, _TPW1)
        def _tile(t):
            tile = wid * _TPW1 + t
            vbase = tile * _T1
            r0 = _sload(rs_v, tile)
            r1 = _sload(rs_v, tile + 1)
            a0 = (r0 // 8) * 8
            n = (r1 - a0 + _K1 - 1) // _K1

            def start_idx(c, s):
                pos = a0 + c * _K1
                pltpu.async_copy(
                    fmidx_hbm.at[pl.ds(pos, _K1)], idx_v[s], si[s])
                pltpu.async_copy(
                    seg_hbm.at[pl.ds(pos, _K1)],
                    seg_v[s].at[pl.ds(0, _K1)], si[s])

            def wait_idx(s):
                pltpu.make_async_copy(
                    fmidx_hbm.at[pl.ds(0, _K1)], idx_v[s], si[s]).wait()
                pltpu.make_async_copy(
                    seg_hbm.at[pl.ds(0, _K1)],
                    seg_v[s].at[pl.ds(0, _K1)], si[s]).wait()

            def start_gather(s, b):
                pltpu.async_copy(xmod_hbm.at[idx_v[s]], rows_v[b], sg[b])

            def wait_gather(s, b):
                pltpu.make_async_copy(
                    xmod_hbm.at[idx_v[s]], rows_v[b], sg[b]).wait()

            def process(c, b, s):
                pos = a0 + c * _K1
                lo = jnp.maximum(r0 - pos, 0)
                hi = jnp.minimum(r1 - pos, _K1)
                mid = lo + ((hi - lo) // 4) * 4

                rel0 = _sload(seg_v[s], lo) - vbase
                init = [rel0] + [
                    acc_v[rel0, pl.ds(j * _L, _L)] for j in range(_DB)]
                ninf = jnp.full((_L,), _NEG_INF, jnp.float32)

                def one_row(i, carry):
                    prev = carry[0]
                    rel = _sload(seg_v[s], i) - vbase
                    fresh = rel != prev
                    out = [rel]
                    for j in range(_DB):
                        a = jnp.where(fresh, ninf, carry[1 + j])
                        a = jnp.maximum(a, rows_v[b][i, pl.ds(j * _L, _L)])
                        acc_v[rel, pl.ds(j * _L, _L)] = a
                        out.append(a)
                    return out

                @pl.loop(lo, mid, step=4, init_carry=init)
                def _row4(i, carry):
                    for u in range(4):
                        carry = one_row(i + u, carry)
                    return carry

                carry2 = init if _row4 is None else _row4

                @pl.loop(mid, hi, init_carry=carry2)
                def _row(i, carry):
                    return one_row(i, carry)

            @pl.loop(0, _T1)
            def _init(v):
                for j in range(_DB):
                    acc_v[v, pl.ds(j * _L, _L)] = jnp.full(
                        (_L,), _NEG_INF, jnp.float32)

            @pl.when(n > 0)
            def _p0():
                start_idx(0, 0)

            @pl.when(n > 1)
            def _p1():
                start_idx(1, 1)

            @pl.when(n > 0)
            def _p2():
                wait_idx(0)
                start_gather(0, 0)

            @pl.loop(0, n, step=4)
            def _quad(c):
                for q in range(4):
                    cq = c + q
                    s, b = q % 4, q % 2
                    s1, b1 = (q + 1) % 4, (q + 1) % 2
                    s2 = (q + 2) % 4

                    @pl.when(cq < n)
                    def _(cq=cq, s=s, b=b, s1=s1, b1=b1, s2=s2):
                        wait_gather(s, b)

                        @pl.when(cq + 1 < n)
                        def _():
                            wait_idx(s1)
                            start_gather(s1, b1)

                        @pl.when(cq + 2 < n)
                        def _():
                            start_idx(cq + 2, s2)

                        process(cq, b, s)

            @pl.loop(0, _T1)
            def _fin(v):
                for j in range(_DB):
                    sl = pl.ds(j * _L, _L)
                    x = acc_v[v, sl]
                    acc_v[v, sl] = jnp.where(
                        x == _NEG_INF, jnp.float32(0.0), x)

            pltpu.sync_copy(acc_v, out_hbm.at[pl.ds(vbase, _T1)])

    return k(x_mod, fm_idx_pad, seg_pad, rs_pad)


def _stage2(x_atomic, vseg_pad, rs_pad):
    @functools.partial(
        pl.kernel,
        out_type=(
            jax.ShapeDtypeStruct((_NPTS_PAD, _D), jnp.float32),
            jax.ShapeDtypeStruct((_NPTS_PAD * _L,), jnp.float32),
        ),
        mesh=_sc_mesh(),
        scratch_types=[
            pltpu.VMEM((rs_pad.shape[0] + _L,), jnp.int32),
            [pltpu.VMEM((_K2 + _L,), jnp.int32)] * 2,
            [pltpu.VMEM((_K2, _D), jnp.float32)] * 2,
            pltpu.VMEM((_PPW, _D), jnp.float32),
            pltpu.VMEM((_PPW * _L,), jnp.int32),
            pltpu.VMEM((_PPW * _L,), jnp.float32),
            [pltpu.SemaphoreType.DMA] * 2,
        ],
    )
    def k(xa_hbm, vseg_hbm, rs_hbm, sums_hbm, cnts_hbm,
          rs_v, seg_v, rows_v, sum_v, last_v, cnt_v, sr):
        wid = lax.axis_index("s") * _NC + lax.axis_index("c")
        pltpu.sync_copy(rs_hbm, rs_v.at[pl.ds(0, rs_hbm.shape[0])])
        p0 = wid * _PPW
        r0 = _sload(rs_v, wid)
        r1 = _sload(rs_v, wid + 1)
        a0 = (r0 // 8) * 8
        n = (r1 - a0 + _K2 - 1) // _K2

        @pl.loop(0, _PPW)
        def _init(v):
            last_v[pl.ds(v * _L, _L)] = jnp.full((_L,), -1, jnp.int32)
            for j in range(_DB):
                sum_v[v, pl.ds(j * _L, _L)] = jnp.zeros((_L,), jnp.float32)

        def start(c, b):
            pos = a0 + c * _K2
            pltpu.async_copy(
                vseg_hbm.at[pl.ds(pos, _K2)],
                seg_v[b].at[pl.ds(0, _K2)], sr[b])
            pltpu.async_copy(xa_hbm.at[pl.ds(pos, _K2)], rows_v[b], sr[b])

        def wait(b):
            pltpu.make_async_copy(
                vseg_hbm.at[pl.ds(0, _K2)],
                seg_v[b].at[pl.ds(0, _K2)], sr[b]).wait()
            pltpu.make_async_copy(
                xa_hbm.at[pl.ds(0, _K2)], rows_v[b], sr[b]).wait()

        def process(c, b):
            pos = a0 + c * _K2
            lo = jnp.maximum(r0 - pos, 0)
            hi = jnp.minimum(r1 - pos, _K2)
            mid = lo + ((hi - lo) // 4) * 4

            rel0 = _sload(seg_v[b], lo) - p0
            init = [rel0] + [
                sum_v[rel0, pl.ds(j * _L, _L)] for j in range(_DB)]
            zero = jnp.zeros((_L,), jnp.float32)

            def one_row(i, carry):
                prev = carry[0]
                rel = _sload(seg_v[b], i) - p0
                fresh = rel != prev
                out = [rel]
                for j in range(_DB):
                    a = jnp.where(fresh, zero, carry[1 + j])
                    a = a + rows_v[b][i, pl.ds(j * _L, _L)]
                    sum_v[rel, pl.ds(j * _L, _L)] = a
                    out.append(a)
                last_v[pl.ds(rel * _L, _L)] = lax.broadcast(pos + i, (_L,))
                return out

            @pl.loop(lo, mid, step=4, init_carry=init)
            def _row4(i, carry):
                for u in range(4):
                    carry = one_row(i + u, carry)
                return carry

            carry2 = init if _row4 is None else _row4

            @pl.loop(mid, hi, init_carry=carry2)
            def _row(i, carry):
                return one_row(i, carry)

        @pl.when(n > 0)
        def _p0():
            start(0, 0)

        @pl.when(n > 1)
        def _p1():
            start(1, 1)

        @pl.loop(0, n, step=2)
        def _duo(c):
            for q in range(2):
                cq = c + q
                b = q % 2

                @pl.when(cq < n)
                def _(cq=cq, b=b):
                    wait(b)

                    @pl.when(cq + 2 < n)
                    def _():
                        start(cq + 2, b)

                    process(cq, b)

        @pl.loop(0, _PPW, init_carry=r0 - 1)
        def _walk(v, prev):
            lp = last_v[pl.ds(v * _L, _L)][0]
            valid = lp >= 0
            cnt = jnp.where(valid, lp - prev, 0)
            cnt_v[pl.ds(v * _L, _L)] = lax.broadcast(
                cnt, (_L,)).astype(jnp.float32)
            return jnp.where(valid, lp, prev)

        pltpu.sync_copy(sum_v, sums_hbm.at[pl.ds(p0, _PPW)])
        pltpu.sync_copy(cnt_v, cnts_hbm.at[pl.ds(p0 * _L, _PPW * _L)])

    return k(x_atomic, vseg_pad, rs_pad)


def _finish_body(x3_ref, s_ref, c_ref, o_ref):
    inv = jnp.float32(1.0) / jnp.maximum(c_ref[...], jnp.float32(1.0))
    o_ref[...] = x3_ref[...] + s_ref[...] * jnp.min(inv, axis=1, keepdims=True)


def _finish(x3_pad, sums, cnts):
    bm = 1280
    return pl.pallas_call(
        _finish_body,
        grid=(_NPTS_PAD // bm,),
        in_specs=[
            pl.BlockSpec((bm, _D), lambda i: (i, 0)),
            pl.BlockSpec((bm, _D), lambda i: (i, 0)),
            pl.BlockSpec((bm, _L), lambda i: (i, 0)),
        ],
        out_specs=pl.BlockSpec((bm, _D), lambda i: (i, 0)),
        out_shape=jax.ShapeDtypeStruct((_NPTS_PAD, _D), jnp.float32),
    )(x3_pad, sums, cnts)


def kernel(x_3d, x_mod_feat, fm_idx, atomic_seg, view_seg, W_conv, b_conv):
    fm = fm_idx.astype(jnp.int32)
    aseg = atomic_seg.astype(jnp.int32)
    vseg = view_seg.astype(jnp.int32)

    x_mod = _conv(x_mod_feat, W_conv, b_conv)

    vb1 = jnp.arange(0, _NV_PAD + 1, _T1, dtype=jnp.int32)
    rs1 = jnp.searchsorted(aseg, vb1).astype(jnp.int32)
    rs1 = jnp.concatenate([rs1, jnp.full((7,), _N_MAP, jnp.int32)])
    fm_pad = jnp.concatenate([fm, jnp.zeros((_K1,), jnp.int32)])
    aseg_pad = jnp.concatenate([aseg, jnp.full((_K1,), _N_VIEWS, jnp.int32)])

    x_atomic = _stage1(x_mod, fm_pad, aseg_pad, rs1)

    pb = jnp.arange(0, _NPTS_PAD + 1, _PPW, dtype=jnp.int32)
    rs2 = jnp.searchsorted(vseg, pb).astype(jnp.int32)
    rs2 = jnp.concatenate([rs2, jnp.full((7,), _N_VIEWS, jnp.int32)])
    vseg_pad = jnp.concatenate([vseg, jnp.full((_K2,), _NPTS_PAD, jnp.int32)])
    x3_pad = jnp.concatenate(
        [x_3d, jnp.zeros((_NPTS_PAD - _N_POINTS, _D), jnp.float32)])

    sums, cnts = _stage2(x_atomic, vseg_pad, rs2)
    out = _finish(x3_pad, sums, cnts.reshape(_NPTS_PAD, _L))
    return out[:_N_POINTS]


# speedup vs baseline: 2.9251x; 1.0002x over previous
import functools

import jax
import jax.numpy as jnp
from jax import lax
from jax.experimental import pallas as pl
from jax.experimental.pallas import tpu as pltpu
from jax.experimental.pallas import tpu_sc as plsc

_N_POINTS = 10000
_N_PIXELS = 50000
_N_MAP = 320000
_N_VIEWS = 40000
_D = 128
_L = 16
_DB = _D // _L

_NC, _NS = 2, 16
_NW = _NC * _NS

_T1 = 640
_NV_PAD = 40960
_NT1 = _NV_PAD // _T1
_TPW1 = _NT1 // _NW
_K1 = 128

_K2 = 256
_PPW = 320
_NPTS_PAD = _PPW * _NW

_NEG_INF = float("-inf")


def _sload(ref, i):
    return ref[pl.ds(i, _L)][0]


def _conv_body(x_ref, w_ref, b_ref, o_ref):
    o_ref[...] = (
        jnp.dot(x_ref[...], w_ref[...], preferred_element_type=jnp.float32)
        + b_ref[...]
    )


def _conv(x, w, b):
    m = x.shape[0]
    bm = 2000
    return pl.pallas_call(
        _conv_body,
        grid=(m // bm,),
        in_specs=[
            pl.BlockSpec((bm, _D), lambda i: (i, 0)),
            pl.BlockSpec((_D, _D), lambda i: (0, 0)),
            pl.BlockSpec((1, _D), lambda i: (0, 0)),
        ],
        out_specs=pl.BlockSpec((bm, _D), lambda i: (i, 0)),
        out_shape=jax.ShapeDtypeStruct((m, _D), jnp.float32),
    )(x, w, b.reshape(1, _D))


def _sc_mesh():
    return plsc.VectorSubcoreMesh(core_axis_name="c", subcore_axis_name="s")


def _stage1(x_mod, fm_idx_pad, seg_pad, rs_pad):
    @functools.partial(
        pl.kernel,
        out_type=jax.ShapeDtypeStruct((_NV_PAD + _K2, _D), jnp.float32),
        mesh=_sc_mesh(),
        scratch_types=[
            pltpu.VMEM((rs_pad.shape[0] + _L,), jnp.int32),
            [pltpu.VMEM((_K1,), jnp.int32)] * 4,
            [pltpu.VMEM((_K1 + _L,), jnp.int32)] * 4,
            [pltpu.VMEM((_K1, _D), jnp.float32)] * 2,
            pltpu.VMEM((_T1, _D), jnp.float32),
            [pltpu.SemaphoreType.DMA] * 4,
            [pltpu.SemaphoreType.DMA] * 2,
        ],
    )
    def k(xmod_hbm, fmidx_hbm, seg_hbm, rs_hbm, out_hbm,
          rs_v, idx_v, seg_v, rows_v, acc_v, si, sg):
        wid = lax.axis_index("s") * _NC + lax.axis_index("c")
        pltpu.sync_copy(rs_hbm, rs_v.at[pl.ds(0, rs_hbm.shape[0])])

        @pl.loop(0, _TPW1)
        def _tile(t):
            tile = wid * _TPW1 + t
            vbase = tile * _T1
            r0 = _sload(rs_v, tile)
            r1 = _sload(rs_v, tile + 1)
            a0 = (r0 // 8) * 8
            n = (r1 - a0 + _K1 - 1) // _K1

            def start_idx(c, s):
                pos = a0 + c * _K1
                pltpu.async_copy(
                    fmidx_hbm.at[pl.ds(pos, _K1)], idx_v[s], si[s])
                pltpu.async_copy(
                    seg_hbm.at[pl.ds(pos, _K1)],
                    seg_v[s].at[pl.ds(0, _K1)], si[s])

            def wait_idx(s):
                pltpu.make_async_copy(
                    fmidx_hbm.at[pl.ds(0, _K1)], idx_v[s], si[s]).wait()
                pltpu.make_async_copy(
                    seg_hbm.at[pl.ds(0, _K1)],
                    seg_v[s].at[pl.ds(0, _K1)], si[s]).wait()

            def start_gather(s, b):
                pltpu.async_copy(xmod_hbm.at[idx_v[s]], rows_v[b], sg[b])

            def wait_gather(s, b):
                pltpu.make_async_copy(
                    xmod_hbm.at[idx_v[s]], rows_v[b], sg[b]).wait()

            def process(c, b, s):
                pos = a0 + c * _K1
                lo = jnp.maximum(r0 - pos, 0)
                hi = jnp.minimum(r1 - pos, _K1)
                mid = lo + ((hi - lo) // 4) * 4

                rel0 = _sload(seg_v[s], lo) - vbase
                init = [rel0] + [
                    acc_v[rel0, pl.ds(j * _L, _L)] for j in range(_DB)]
                ninf = jnp.full((_L,), _NEG_INF, jnp.float32)

                def one_row(i, carry):
                    prev = carry[0]
                    rel = _sload(seg_v[s], i) - vbase
                    fresh = rel != prev
                    out = [rel]
                    for j in range(_DB):
                        a = jnp.where(fresh, ninf, carry[1 + j])
                        a = jnp.maximum(a, rows_v[b][i, pl.ds(j * _L, _L)])
                        acc_v[rel, pl.ds(j * _L, _L)] = a
                        out.append(a)
                    return out

                @pl.loop(lo, mid, step=4, init_carry=init)
                def _row4(i, carry):
                    for u in range(4):
                        carry = one_row(i + u, carry)
                    return carry

                carry2 = init if _row4 is None else _row4

                @pl.loop(mid, hi, init_carry=carry2)
                def _row(i, carry):
                    return one_row(i, carry)

            @pl.loop(0, _T1)
            def _init(v):
                for j in range(_DB):
                    acc_v[v, pl.ds(j * _L, _L)] = jnp.full(
                        (_L,), _NEG_INF, jnp.float32)

            @pl.when(n > 0)
            def _p0():
                start_idx(0, 0)

            @pl.when(n > 1)
            def _p1():
                start_idx(1, 1)

            @pl.when(n > 0)
            def _p2():
                wait_idx(0)
                start_gather(0, 0)

            @pl.loop(0, n, step=4)
            def _quad(c):
                for q in range(4):
                    cq = c + q
                    s, b = q % 4, q % 2
                    s1, b1 = (q + 1) % 4, (q + 1) % 2
                    s2 = (q + 2) % 4

                    @pl.when(cq < n)
                    def _(cq=cq, s=s, b=b, s1=s1, b1=b1, s2=s2):
                        wait_gather(s, b)

                        @pl.when(cq + 1 < n)
                        def _():
                            wait_idx(s1)
                            start_gather(s1, b1)

                        @pl.when(cq + 2 < n)
                        def _():
                            start_idx(cq + 2, s2)

                        process(cq, b, s)

            @pl.loop(0, _T1)
            def _fin(v):
                for j in range(_DB):
                    sl = pl.ds(j * _L, _L)
                    x = acc_v[v, sl]
                    acc_v[v, sl] = jnp.where(
                        x == _NEG_INF, jnp.float32(0.0), x)

            pltpu.sync_copy(acc_v, out_hbm.at[pl.ds(vbase, _T1)])

    return k(x_mod, fm_idx_pad, seg_pad, rs_pad)


def _stage2(x_atomic, vseg_pad, rs_pad):
    @functools.partial(
        pl.kernel,
        out_type=(
            jax.ShapeDtypeStruct((_NPTS_PAD, _D), jnp.float32),
            jax.ShapeDtypeStruct((_NPTS_PAD * _L,), jnp.float32),
        ),
        mesh=_sc_mesh(),
        scratch_types=[
            pltpu.VMEM((rs_pad.shape[0] + _L,), jnp.int32),
            [pltpu.VMEM((_K2 + _L,), jnp.int32)] * 2,
            [pltpu.VMEM((_K2, _D), jnp.float32)] * 2,
            pltpu.VMEM((_PPW, _D), jnp.float32),
            pltpu.VMEM((_PPW * _L,), jnp.int32),
            pltpu.VMEM((_PPW * _L,), jnp.float32),
            [pltpu.SemaphoreType.DMA] * 2,
        ],
    )
    def k(xa_hbm, vseg_hbm, rs_hbm, sums_hbm, cnts_hbm,
          rs_v, seg_v, rows_v, sum_v, last_v, cnt_v, sr):
        wid = lax.axis_index("s") * _NC + lax.axis_index("c")
        pltpu.sync_copy(rs_hbm, rs_v.at[pl.ds(0, rs_hbm.shape[0])])
        p0 = wid * _PPW
        r0 = _sload(rs_v, wid)
        r1 = _sload(rs_v, wid + 1)
        a0 = (r0 // 8) * 8
        n = (r1 - a0 + _K2 - 1) // _K2

        @pl.loop(0, _PPW)
        def _init(v):
            last_v[pl.ds(v * _L, _L)] = jnp.full((_L,), -1, jnp.int32)
            for j in range(_DB):
                sum_v[v, pl.ds(j * _L, _L)] = jnp.zeros((_L,), jnp.float32)

        def start(c, b):
            pos = a0 + c * _K2
            pltpu.async_copy(
                vseg_hbm.at[pl.ds(pos, _K2)],
                seg_v[b].at[pl.ds(0, _K2)], sr[b])
            pltpu.async_copy(xa_hbm.at[pl.ds(pos, _K2)], rows_v[b], sr[b])

        def wait(b):
            pltpu.make_async_copy(
                vseg_hbm.at[pl.ds(0, _K2)],
                seg_v[b].at[pl.ds(0, _K2)], sr[b]).wait()
            pltpu.make_async_copy(
                xa_hbm.at[pl.ds(0, _K2)], rows_v[b], sr[b]).wait()

        def process(c, b):
            pos = a0 + c * _K2
            lo = jnp.maximum(r0 - pos, 0)
            hi = jnp.minimum(r1 - pos, _K2)
            mid = lo + ((hi - lo) // 4) * 4

            rel0 = _sload(seg_v[b], lo) - p0
            init = [rel0] + [
                sum_v[rel0, pl.ds(j * _L, _L)] for j in range(_DB)]
            zero = jnp.zeros((_L,), jnp.float32)

            def one_row(i, carry):
                prev = carry[0]
                rel = _sload(seg_v[b], i) - p0
                fresh = rel != prev
                out = [rel]
                for j in range(_DB):
                    a = jnp.where(fresh, zero, carry[1 + j])
                    a = a + rows_v[b][i, pl.ds(j * _L, _L)]
                    sum_v[rel, pl.ds(j * _L, _L)] = a
                    out.append(a)
                last_v[pl.ds(rel * _L, _L)] = lax.broadcast(pos + i, (_L,))
                return out

            @pl.loop(lo, mid, step=4, init_carry=init)
            def _row4(i, carry):
                for u in range(4):
                    carry = one_row(i + u, carry)
                return carry

            carry2 = init if _row4 is None else _row4

            @pl.loop(mid, hi, init_carry=carry2)
            def _row(i, carry):
                return one_row(i, carry)

        @pl.when(n > 0)
        def _p0():
            start(0, 0)

        @pl.when(n > 1)
        def _p1():
            start(1, 1)

        @pl.loop(0, n, step=2)
        def _duo(c):
            for q in range(2):
                cq = c + q
                b = q % 2

                @pl.when(cq < n)
                def _(cq=cq, b=b):
                    wait(b)
                    process(cq, b)

                    @pl.when(cq + 2 < n)
                    def _():
                        start(cq + 2, b)

        @pl.loop(0, _PPW, init_carry=r0 - 1)
        def _walk(v, prev):
            lp = last_v[pl.ds(v * _L, _L)][0]
            valid = lp >= 0
            cnt = jnp.where(valid, lp - prev, 0)
            cnt_v[pl.ds(v * _L, _L)] = lax.broadcast(
                cnt, (_L,)).astype(jnp.float32)
            return jnp.where(valid, lp, prev)

        pltpu.sync_copy(sum_v, sums_hbm.at[pl.ds(p0, _PPW)])
        pltpu.sync_copy(cnt_v, cnts_hbm.at[pl.ds(p0 * _L, _PPW * _L)])

    return k(x_atomic, vseg_pad, rs_pad)


def _finish_body(x3_ref, s_ref, c_ref, o_ref):
    inv = jnp.float32(1.0) / jnp.maximum(c_ref[...], jnp.float32(1.0))
    o_ref[...] = x3_ref[...] + s_ref[...] * jnp.min(inv, axis=1, keepdims=True)


def _finish(x3_pad, sums, cnts):
    bm = 1280
    return pl.pallas_call(
        _finish_body,
        grid=(_NPTS_PAD // bm,),
        in_specs=[
            pl.BlockSpec((bm, _D), lambda i: (i, 0)),
            pl.BlockSpec((bm, _D), lambda i: (i, 0)),
            pl.BlockSpec((bm, _L), lambda i: (i, 0)),
        ],
        out_specs=pl.BlockSpec((bm, _D), lambda i: (i, 0)),
        out_shape=jax.ShapeDtypeStruct((_NPTS_PAD, _D), jnp.float32),
    )(x3_pad, sums, cnts)


def kernel(x_3d, x_mod_feat, fm_idx, atomic_seg, view_seg, W_conv, b_conv):
    fm = fm_idx.astype(jnp.int32)
    aseg = atomic_seg.astype(jnp.int32)
    vseg = view_seg.astype(jnp.int32)

    x_mod = _conv(x_mod_feat, W_conv, b_conv)

    vb1 = jnp.arange(0, _NV_PAD + 1, _T1, dtype=jnp.int32)
    rs1 = jnp.searchsorted(aseg, vb1).astype(jnp.int32)
    rs1 = jnp.concatenate([rs1, jnp.full((7,), _N_MAP, jnp.int32)])
    fm_pad = jnp.concatenate([fm, jnp.zeros((_K1,), jnp.int32)])
    aseg_pad = jnp.concatenate([aseg, jnp.full((_K1,), _N_VIEWS, jnp.int32)])

    x_atomic = _stage1(x_mod, fm_pad, aseg_pad, rs1)

    pb = jnp.arange(0, _NPTS_PAD + 1, _PPW, dtype=jnp.int32)
    rs2 = jnp.searchsorted(vseg, pb).astype(jnp.int32)
    rs2 = jnp.concatenate([rs2, jnp.full((7,), _N_VIEWS, jnp.int32)])
    vseg_pad = jnp.concatenate([vseg, jnp.full((_K2,), _NPTS_PAD, jnp.int32)])
    x3_pad = jnp.concatenate(
        [x_3d, jnp.zeros((_NPTS_PAD - _N_POINTS, _D), jnp.float32)])

    sums, cnts = _stage2(x_atomic, vseg_pad, rs2)
    out = _finish(x3_pad, sums, cnts.reshape(_NPTS_PAD, _L))
    return out[:_N_POINTS]
